# baseline (speedup 1.0000x reference)
import jax
import jax.numpy as jnp
from jax.experimental import pallas as pl
from jax.experimental.pallas import tpu as pltpu

_ALPHA = 0.1
_BM = 512


def _body(adj_ref, x_ref, h_ref, o_ref):
    acc = jnp.dot(adj_ref[...], x_ref[...], preferred_element_type=jnp.float32)
    o_ref[...] = (1.0 - _ALPHA) * acc + _ALPHA * h_ref[...]


def kernel(x, adj, h):
    n, d = x.shape
    return pl.pallas_call(
        _body,
        grid=(n // _BM,),
        in_specs=[
            pl.BlockSpec((_BM, n), lambda i: (i, 0)),
            pl.BlockSpec((n, d), lambda i: (0, 0)),
            pl.BlockSpec((_BM, d), lambda i: (i, 0)),
        ],
        out_specs=pl.BlockSpec((_BM, d), lambda i: (i, 0)),
        out_shape=jax.ShapeDtypeStruct((n, d), x.dtype),
        compiler_params=pltpu.CompilerParams(dimension_semantics=("parallel",)),
    )(adj, x, h)


# speedup vs baseline: 1.0034x; 1.0034x over previous
import jax
import jax.numpy as jnp
from jax.experimental import pallas as pl
from jax.experimental.pallas import tpu as pltpu

_ALPHA = 0.1
_BM = 512


def _body(adj_ref, x_ref, h_ref, o_ref):
    a = adj_ref[...].astype(jnp.bfloat16)
    b = x_ref[...].astype(jnp.bfloat16)
    acc = jnp.dot(a, b, preferred_element_type=jnp.float32)
    o_ref[...] = (1.0 - _ALPHA) * acc + _ALPHA * h_ref[...]


def kernel(x, adj, h):
    n, d = x.shape
    return pl.pallas_call(
        _body,
        grid=(n // _BM,),
        in_specs=[
            pl.BlockSpec((_BM, n), lambda i: (i, 0)),
            pl.BlockSpec((n, d), lambda i: (0, 0)),
            pl.BlockSpec((_BM, d), lambda i: (i, 0)),
        ],
        out_specs=pl.BlockSpec((_BM, d), lambda i: (i, 0)),
        out_shape=jax.ShapeDtypeStruct((n, d), x.dtype),
        compiler_params=pltpu.CompilerParams(dimension_semantics=("parallel",)),
    )(adj, x, h)
